# baseline (speedup 1.0000x reference)
import jax
import jax.numpy as jnp
from jax.experimental import pallas as pl
from jax.experimental.pallas import tpu as pltpu

_TAU = 0.3
_BB = 8
_CH = 4096
_LOG2E = 1.4426950408889634


def _body(logits_ref, u_ref, out_ref, x2_ref):
    lgt2 = logits_ref[...] * (_LOG2E / _TAU)
    D = lgt2.shape[1]
    nch = D // _CH
    m2 = None
    for c in range(nch):
        sl = slice(c * _CH, (c + 1) * _CH)
        u = u_ref[:, :, sl]
        w = -jnp.log(u)
        x2 = lgt2[:, None, sl] - jnp.log(w) * (_LOG2E / _TAU)
        x2_ref[:, :, sl] = x2
        pm = jnp.max(x2, axis=2, keepdims=True)
        m2 = pm if m2 is None else jnp.maximum(m2, pm)
    s = None
    for c in range(nch):
        sl = slice(c * _CH, (c + 1) * _CH)
        e = jnp.exp2(x2_ref[:, :, sl] - m2)
        ps = jnp.sum(e, axis=2, keepdims=True)
        s = ps if s is None else s + ps
    c2 = m2 + jnp.log2(s)
    for c in range(nch):
        sl = slice(c * _CH, (c + 1) * _CH)
        z = x2_ref[:, :, sl] - c2
        out_ref[:, sl] = jnp.exp2(jnp.max(z.transpose(1, 0, 2), axis=0))


def kernel(logits, uniform):
    B, D = logits.shape
    K = uniform.shape[1]
    return pl.pallas_call(
        _body,
        grid=(B // _BB,),
        in_specs=[
            pl.BlockSpec((_BB, D), lambda b: (b, 0)),
            pl.BlockSpec((_BB, K, D), lambda b: (b, 0, 0)),
        ],
        out_specs=pl.BlockSpec((_BB, D), lambda b: (b, 0)),
        out_shape=jax.ShapeDtypeStruct((B, D), jnp.float32),
        scratch_shapes=[pltpu.VMEM((_BB, K, D), jnp.float32)],
        compiler_params=pltpu.CompilerParams(
            dimension_semantics=("parallel",)),
    )(logits, uniform)
